# Initial kernel scaffold; baseline (speedup 1.0000x reference)
import jax
import jax.numpy as jnp
from jax.experimental import pallas as pl


def kernel(fm1, edges_f, edges_s, edges_g, dm_f, dm_s, dm_g, W_x1_f, b_x1_f, W_x2_f, b_x2_f, W_x1_s, b_x1_s, W_x2_s, b_x2_s, W_x1_g, b_x1_g, W_x2_g, b_x2_g, fc1_W, fc1_b, fc2_W, fc2_b, cnn_w, cnn_b):
    raise NotImplementedError("write your pallas kernel here")


# speedup vs baseline: 53.1286x; 53.1286x over previous
import functools

import jax
import jax.numpy as jnp
from jax import lax
from jax.experimental import pallas as pl
from jax.experimental.pallas import tpu as pltpu, tpu_sc as plsc

N = 853
FM = 512
E = 54592

NP = 896
NN = NP * NP
DMP = N * N + 7
EPT = 3456
EPAD = 16 * EPT
CHUNK = 128
NCH = EPT // CHUNK
ROWS = EPAD // CHUNK
SLICE = NN // 16
ZCH = SLICE // 16

_f32 = jnp.float32
_i32 = jnp.int32


def _sc_body(gi_f, si_f, gi_s, si_s, gi_g, si_g, dm_f, dm_s, dm_g,
             a_f, a_s, a_g, gbuf, sbuf, ewbuf, zbuf, acc0, acc1, sem):
    cid = lax.axis_index("c")
    sid = lax.axis_index("s")

    def zero_zbuf(i, c):
        zbuf[pl.ds(i * 16, 16)] = jnp.zeros((16,), _f32)
        return c

    lax.fori_loop(0, ZCH // 16, zero_zbuf, 0)

    base = sid * SLICE
    for k in range(16):
        pltpu.sync_copy(zbuf, acc0.at[pl.ds(base + k * ZCH, ZCH)])

    @pl.when(cid == 0)
    def _():
        for k in range(16):
            pltpu.sync_copy(zbuf, acc1.at[pl.ds(base + k * ZCH, ZCH)])

    plsc.subcore_barrier()

    def run_view(gi, si, dm, acc):
        pltpu.sync_copy(gi.at[sid], gbuf)
        pltpu.sync_copy(si.at[sid], sbuf)
        cps = [pltpu.async_copy(dm.at[gbuf.at[j]], ewbuf.at[j], sem)
               for j in range(NCH)]
        for c in cps:
            c.wait()
        for j in range(NCH):
            pltpu.sync_copy(ewbuf.at[j], acc.at[sbuf.at[j]], add=True)

    @pl.when(cid == 0)
    def _():
        run_view(gi_f, si_f, dm_f, acc0)
        run_view(gi_s, si_s, dm_s, acc1)

    @pl.when(cid == 1)
    def _():
        run_view(gi_g, si_g, dm_g, acc0)

    plsc.subcore_barrier()

    sl = pl.ds(base, SLICE)

    @pl.when(cid == 0)
    def _():
        pltpu.sync_copy(acc0.at[sl], a_f.at[sl])
        pltpu.sync_copy(acc1.at[sl], a_s.at[sl])

    @pl.when(cid == 1)
    def _():
        pltpu.sync_copy(acc0.at[sl], a_g.at[sl])


@functools.cache
def _sc_build():
    return pl.kernel(
        _sc_body,
        out_type=[jax.ShapeDtypeStruct((NN,), _f32)] * 3,
        mesh=plsc.VectorSubcoreMesh(core_axis_name="c", subcore_axis_name="s"),
        scratch_types=[
            pltpu.VMEM((NCH, CHUNK), _i32),
            pltpu.VMEM((NCH, CHUNK), _i32),
            pltpu.VMEM((NCH, CHUNK), _f32),
            pltpu.VMEM((ZCH,), _f32),
            pltpu.VMEM_SHARED((NN,), _f32),
            pltpu.VMEM_SHARED((NN,), _f32),
            pltpu.SemaphoreType.DMA,
        ],
    )


def _sigmoid(x):
    return 1.0 / (1.0 + jnp.exp(-x))


def _tc_body(fm1_ref, af_ref, as_ref, ag_ref,
             w1f, b1f, w2f, b2f, w1s, b1s, w2s, b2s, w1g, b1g, w2g, b2g,
             fc1w, fc1b, fc2w, fc2b, cw, cb, rmask, out_ref):
    fm1 = fm1_ref[...]
    mask = rmask[...]
    xs = []
    for a_ref, w1, b1, w2, b2 in (
            (af_ref, w1f, b1f, w2f, b2f),
            (as_ref, w1s, b1s, w2s, b2s),
            (ag_ref, w1g, b1g, w2g, b2g)):
        A = a_ref[...]
        deg = jnp.sum(A, axis=1, keepdims=True) + 1.0
        dis = lax.rsqrt(deg) * mask
        z1 = dis * jnp.dot(fm1, w1[...], preferred_element_type=_f32)
        y1 = jnp.dot(A, z1, preferred_element_type=_f32) + z1
        x1 = jnp.maximum(dis * y1 + b1[...], 0.0) * mask
        z2 = dis * jnp.dot(x1, w2[...], preferred_element_type=_f32)
        y2 = jnp.dot(A, z2, preferred_element_type=_f32) + z2
        x2 = jnp.maximum(dis * y2 + b2[...], 0.0) * mask
        xs.append(x1)
        xs.append(x2)

    inv = 1.0 / (N * FM)
    att1 = fc1b[...]
    for c in range(6):
        att1 = att1 + (jnp.sum(xs[c]) * inv) * fc1w[c:c + 1, :]
    att1 = jnp.maximum(att1, 0.0)
    att2 = _sigmoid(jnp.dot(att1, fc2w[...],
                            preferred_element_type=_f32) + fc2b[...])
    s = att2 * cw[...]

    acc = jnp.full((NP, FM), cb[0, 0], _f32)
    for c in range(6):
        acc = acc + s[0:1, c:c + 1] * xs[c]
    out_ref[...] = acc


_tc_call = pl.pallas_call(
    _tc_body,
    out_shape=jax.ShapeDtypeStruct((NP, FM), _f32),
)


def _edge_indices(edges):
    src = edges[0].astype(_i32)
    dst = edges[1].astype(_i32)
    gidx = jnp.concatenate(
        [src * N + dst, jnp.full((EPAD - E,), N * N, _i32)])
    sidx = jnp.concatenate(
        [dst * NP + src, jnp.full((EPAD - E,), NN - 1, _i32)])
    return (gidx.reshape(16, NCH, CHUNK), sidx.reshape(16, NCH, CHUNK))


def _pad_dm(dm):
    return jnp.concatenate([dm.reshape(-1), jnp.zeros((DMP - N * N,), _f32)])


def kernel(fm1, edges_f, edges_s, edges_g, dm_f, dm_s, dm_g,
           W_x1_f, b_x1_f, W_x2_f, b_x2_f, W_x1_s, b_x1_s, W_x2_s, b_x2_s,
           W_x1_g, b_x1_g, W_x2_g, b_x2_g,
           fc1_W, fc1_b, fc2_W, fc2_b, cnn_w, cnn_b):
    gi_f, si_f = _edge_indices(edges_f)
    gi_s, si_s = _edge_indices(edges_s)
    gi_g, si_g = _edge_indices(edges_g)

    a_f, a_s, a_g = _sc_build()(gi_f, si_f, gi_s, si_s, gi_g, si_g,
                                _pad_dm(dm_f), _pad_dm(dm_s), _pad_dm(dm_g))

    fm1p = jnp.pad(fm1, ((0, NP - N), (0, 0)))
    rmask = (jnp.arange(NP) < N).astype(_f32).reshape(NP, 1)
    fc1w = jnp.pad(fc1_W, ((0, 2), (0, 2)))
    fc1b = jnp.pad(fc1_b, (0, 2)).reshape(1, 32)
    fc2w = jnp.pad(fc2_W, ((0, 2), (0, 2)))
    fc2b = jnp.pad(fc2_b, (0, 2)).reshape(1, 8)
    cw = jnp.pad(cnn_w, (0, 2)).reshape(1, 8)
    cb = cnn_b.reshape(1, 1)

    outp = _tc_call(
        fm1p, a_f.reshape(NP, NP), a_s.reshape(NP, NP), a_g.reshape(NP, NP),
        W_x1_f, b_x1_f.reshape(1, FM), W_x2_f, b_x2_f.reshape(1, FM),
        W_x1_s, b_x1_s.reshape(1, FM), W_x2_s, b_x2_s.reshape(1, FM),
        W_x1_g, b_x1_g.reshape(1, FM), W_x2_g, b_x2_g.reshape(1, FM),
        fc1w, fc1b, fc2w, fc2b, cw, cb, rmask)
    return outp[:N]


# speedup vs baseline: 53.9010x; 1.0145x over previous
import functools

import jax
import jax.numpy as jnp
from jax import lax
from jax.experimental import pallas as pl
from jax.experimental.pallas import tpu as pltpu, tpu_sc as plsc

N = 853
FM = 512
E = 54592

NP = 896
NN = NP * NP
CHUNK = 128
NCHA = 27
NCHS = 14
EPADA = 16 * NCHA * CHUNK
EPADS = 32 * NCHS * CHUNK
SLICE = NN // 16
ZCH = SLICE // 16
GPAD = N * N - 1
SPAD = NN - 1

_f32 = jnp.float32
_i32 = jnp.int32


def _sc_body(gi_f, si_f, gi_g, si_g, gi_s, si_s, dm_f, dm_s, dm_g,
             a_f, a_g, a_s0, a_s1,
             gbA, sbA, ewA, gbS, sbS, ewS, zbuf, acc0, acc1, sem):
    cid = lax.axis_index("c")
    sid = lax.axis_index("s")
    base = sid * SLICE
    sl = pl.ds(base, SLICE)

    def run_core(gi_a, si_a, dm_a, dm_s_, out_a, out_s):
        w = cid * 16 + sid
        pltpu.sync_copy(gi_a.at[sid], gbA)
        pltpu.sync_copy(si_a.at[sid], sbA)
        pltpu.sync_copy(gi_s.at[w], gbS)
        pltpu.sync_copy(si_s.at[w], sbS)
        cps = [pltpu.async_copy(dm_a.at[gbA.at[j]], ewA.at[j], sem)
               for j in range(NCHA)]
        cps += [pltpu.async_copy(dm_s_.at[gbS.at[j]], ewS.at[j], sem)
                for j in range(NCHS)]

        def zero_zbuf(i, c):
            zbuf[pl.ds(i * 16, 16)] = jnp.zeros((16,), _f32)
            return c

        lax.fori_loop(0, ZCH // 16, zero_zbuf, 0)
        for k in range(16):
            pltpu.sync_copy(zbuf, acc0.at[pl.ds(base + k * ZCH, ZCH)])
        for k in range(16):
            pltpu.sync_copy(zbuf, acc1.at[pl.ds(base + k * ZCH, ZCH)])
        plsc.subcore_barrier()

        for c in cps:
            c.wait()
        for j in range(NCHA):
            pltpu.sync_copy(ewA.at[j], acc0.at[sbA.at[j]], add=True)
        for j in range(NCHS):
            pltpu.sync_copy(ewS.at[j], acc1.at[sbS.at[j]], add=True)
        plsc.subcore_barrier()

        pltpu.sync_copy(acc0.at[sl], out_a.at[sl])
        pltpu.sync_copy(acc1.at[sl], out_s.at[sl])

    @pl.when(cid == 0)
    def _():
        run_core(gi_f, si_f, dm_f, dm_s, a_f, a_s0)

    @pl.when(cid == 1)
    def _():
        run_core(gi_g, si_g, dm_g, dm_s, a_g, a_s1)


@functools.cache
def _sc_build():
    return pl.kernel(
        _sc_body,
        out_type=[jax.ShapeDtypeStruct((NN,), _f32)] * 4,
        mesh=plsc.VectorSubcoreMesh(core_axis_name="c", subcore_axis_name="s"),
        scratch_types=[
            pltpu.VMEM((NCHA, CHUNK), _i32),
            pltpu.VMEM((NCHA, CHUNK), _i32),
            pltpu.VMEM((NCHA, CHUNK), _f32),
            pltpu.VMEM((NCHS, CHUNK), _i32),
            pltpu.VMEM((NCHS, CHUNK), _i32),
            pltpu.VMEM((NCHS, CHUNK), _f32),
            pltpu.VMEM((ZCH,), _f32),
            pltpu.VMEM_SHARED((NN,), _f32),
            pltpu.VMEM_SHARED((NN,), _f32),
            pltpu.SemaphoreType.DMA,
        ],
    )


def _sigmoid(x):
    return 1.0 / (1.0 + jnp.exp(-x))


def _tc_body(fm1_ref, af_ref, as0_ref, as1_ref, ag_ref,
             w1f, b1f, w2f, b2f, w1s, b1s, w2s, b2s, w1g, b1g, w2g, b2g,
             fc1w, fc1b, fc2w, fc2b, cw, cb, rmask, out_ref):
    fm1 = fm1_ref[...]
    mask = rmask[...]
    xs = []
    for a_ref, w1, b1, w2, b2 in (
            (af_ref, w1f, b1f, w2f, b2f),
            ((as0_ref, as1_ref), w1s, b1s, w2s, b2s),
            (ag_ref, w1g, b1g, w2g, b2g)):
        if isinstance(a_ref, tuple):
            A = a_ref[0][...] + a_ref[1][...]
        else:
            A = a_ref[...]
        deg = jnp.sum(A, axis=1, keepdims=True) + 1.0
        dis = jnp.where(deg > 0, lax.rsqrt(deg), 0.0) * mask
        z1 = dis * jnp.dot(fm1, w1[...], preferred_element_type=_f32)
        y1 = jnp.dot(A, z1, preferred_element_type=_f32) + z1
        x1 = jnp.maximum(dis * y1 + b1[...], 0.0) * mask
        z2 = dis * jnp.dot(x1, w2[...], preferred_element_type=_f32)
        y2 = jnp.dot(A, z2, preferred_element_type=_f32) + z2
        x2 = jnp.maximum(dis * y2 + b2[...], 0.0) * mask
        xs.append(x1)
        xs.append(x2)

    inv = 1.0 / (N * FM)
    att1 = fc1b[...]
    for c in range(6):
        att1 = att1 + (jnp.sum(xs[c]) * inv) * fc1w[c:c + 1, :]
    att1 = jnp.maximum(att1, 0.0)
    att2 = _sigmoid(jnp.dot(att1, fc2w[...],
                            preferred_element_type=_f32) + fc2b[...])
    s = att2 * cw[...]

    acc = jnp.full((NP, FM), cb[0, 0], _f32)
    for c in range(6):
        acc = acc + s[0:1, c:c + 1] * xs[c]
    out_ref[...] = acc


_tc_call = pl.pallas_call(
    _tc_body,
    out_shape=jax.ShapeDtypeStruct((NP, FM), _f32),
)


def _edge_indices(edges, epad, tiles):
    src = edges[0].astype(_i32)
    dst = edges[1].astype(_i32)
    gidx = jnp.concatenate(
        [src * N + dst, jnp.full((epad - E,), GPAD, _i32)])
    sidx = jnp.concatenate(
        [dst * NP + src, jnp.full((epad - E,), SPAD, _i32)])
    nch = epad // (tiles * CHUNK)
    return (gidx.reshape(tiles, nch, CHUNK), sidx.reshape(tiles, nch, CHUNK))


def kernel(fm1, edges_f, edges_s, edges_g, dm_f, dm_s, dm_g,
           W_x1_f, b_x1_f, W_x2_f, b_x2_f, W_x1_s, b_x1_s, W_x2_s, b_x2_s,
           W_x1_g, b_x1_g, W_x2_g, b_x2_g,
           fc1_W, fc1_b, fc2_W, fc2_b, cnn_w, cnn_b):
    gi_f, si_f = _edge_indices(edges_f, EPADA, 16)
    gi_s, si_s = _edge_indices(edges_s, EPADS, 32)
    gi_g, si_g = _edge_indices(edges_g, EPADA, 16)

    a_f, a_g, a_s0, a_s1 = _sc_build()(
        gi_f, si_f, gi_g, si_g, gi_s, si_s,
        dm_f.reshape(-1), dm_s.reshape(-1), dm_g.reshape(-1))

    fm1p = jnp.pad(fm1, ((0, NP - N), (0, 0)))
    rmask = (jnp.arange(NP) < N).astype(_f32).reshape(NP, 1)
    fc1w = jnp.pad(fc1_W, ((0, 2), (0, 2)))
    fc1b = jnp.pad(fc1_b, (0, 2)).reshape(1, 32)
    fc2w = jnp.pad(fc2_W, ((0, 2), (0, 2)))
    fc2b = jnp.pad(fc2_b, (0, 2)).reshape(1, 8)
    cw = jnp.pad(cnn_w, (0, 2)).reshape(1, 8)
    cb = cnn_b.reshape(1, 1)

    outp = _tc_call(
        fm1p, a_f.reshape(NP, NP), a_s0.reshape(NP, NP),
        a_s1.reshape(NP, NP), a_g.reshape(NP, NP),
        W_x1_f, b_x1_f.reshape(1, FM), W_x2_f, b_x2_f.reshape(1, FM),
        W_x1_s, b_x1_s.reshape(1, FM), W_x2_s, b_x2_s.reshape(1, FM),
        W_x1_g, b_x1_g.reshape(1, FM), W_x2_g, b_x2_g.reshape(1, FM),
        fc1w, fc1b, fc2w, fc2b, cw, cb, rmask)
    return outp[:N]


# speedup vs baseline: 54.9545x; 1.0195x over previous
import functools

import jax
import jax.numpy as jnp
from jax import lax
from jax.experimental import pallas as pl
from jax.experimental.pallas import tpu as pltpu, tpu_sc as plsc

N = 853
FM = 512
E = 54592

NP = 896
NN = NP * NP
CHUNK = 128
NCHA = 27
NCHS = 14
EPADA = 16 * NCHA * CHUNK
EPADS = 32 * NCHS * CHUNK
SLICE = NN // 16
ZCH = SLICE // 16
GPAD = N * N - 1
SPAD = NN - 1

_f32 = jnp.float32
_i32 = jnp.int32


def _sc_body(gi_f, si_f, gi_g, si_g, gi_s, si_s, dm_f, dm_s, dm_g,
             a_f, a_g, a_s0, a_s1,
             gbA, sbA, ewA, gbS, sbS, ewS, zbuf, acc0, acc1, sem):
    cid = lax.axis_index("c")
    sid = lax.axis_index("s")
    base = sid * SLICE
    sl = pl.ds(base, SLICE)

    def run_core(gi_a, si_a, dm_a, dm_s_, out_a, out_s):
        w = cid * 16 + sid
        pltpu.sync_copy(gi_a.at[sid], gbA)
        pltpu.sync_copy(si_a.at[sid], sbA)
        pltpu.sync_copy(gi_s.at[w], gbS)
        pltpu.sync_copy(si_s.at[w], sbS)
        cps = [pltpu.async_copy(dm_a.at[gbA.at[j]], ewA.at[j], sem)
               for j in range(NCHA)]
        cps += [pltpu.async_copy(dm_s_.at[gbS.at[j]], ewS.at[j], sem)
                for j in range(NCHS)]

        def zero_zbuf(i, c):
            zbuf[pl.ds(i * 16, 16)] = jnp.zeros((16,), _f32)
            return c

        lax.fori_loop(0, ZCH // 16, zero_zbuf, 0)
        for k in range(16):
            pltpu.sync_copy(zbuf, acc0.at[pl.ds(base + k * ZCH, ZCH)])
        for k in range(16):
            pltpu.sync_copy(zbuf, acc1.at[pl.ds(base + k * ZCH, ZCH)])
        plsc.subcore_barrier()

        for c in cps:
            c.wait()
        for j in range(NCHA):
            pltpu.sync_copy(ewA.at[j], acc0.at[sbA.at[j]], add=True)
        for j in range(NCHS):
            pltpu.sync_copy(ewS.at[j], acc1.at[sbS.at[j]], add=True)
        plsc.subcore_barrier()

        pltpu.sync_copy(acc0.at[sl], out_a.at[sl])
        pltpu.sync_copy(acc1.at[sl], out_s.at[sl])

    @pl.when(cid == 0)
    def _():
        run_core(gi_f, si_f, dm_f, dm_s, a_f, a_s0)

    @pl.when(cid == 1)
    def _():
        run_core(gi_g, si_g, dm_g, dm_s, a_g, a_s1)


@functools.cache
def _sc_build():
    return pl.kernel(
        _sc_body,
        out_type=[jax.ShapeDtypeStruct((NN,), _f32)] * 4,
        mesh=plsc.VectorSubcoreMesh(core_axis_name="c", subcore_axis_name="s"),
        scratch_types=[
            pltpu.VMEM((NCHA, CHUNK), _i32),
            pltpu.VMEM((NCHA, CHUNK), _i32),
            pltpu.VMEM((NCHA, CHUNK), _f32),
            pltpu.VMEM((NCHS, CHUNK), _i32),
            pltpu.VMEM((NCHS, CHUNK), _i32),
            pltpu.VMEM((NCHS, CHUNK), _f32),
            pltpu.VMEM((ZCH,), _f32),
            pltpu.VMEM_SHARED((NN,), _f32),
            pltpu.VMEM_SHARED((NN,), _f32),
            pltpu.SemaphoreType.DMA,
        ],
    )


def _sigmoid(x):
    return 1.0 / (1.0 + jnp.exp(-x))


def _tc_body(fm1_ref, af_ref, as0_ref, as1_ref, ag_ref,
             w1f, b1f, w2f, b2f, w1s, b1s, w2s, b2s, w1g, b1g, w2g, b2g,
             fc1w, fc1b, fc2w, fc2b, cw, cb, rmask, out_ref):
    fm1 = fm1_ref[...]
    mask = rmask[...]
    xs = []
    for a_ref, w1, b1, w2, b2 in (
            (af_ref, w1f, b1f, w2f, b2f),
            ((as0_ref, as1_ref), w1s, b1s, w2s, b2s),
            (ag_ref, w1g, b1g, w2g, b2g)):
        if isinstance(a_ref, tuple):
            A = a_ref[0][...] + a_ref[1][...]
        else:
            A = a_ref[...]
        deg = jnp.sum(A, axis=1, keepdims=True) + 1.0
        dis = jnp.where(deg > 0, lax.rsqrt(deg), 0.0) * mask
        Ab = A.astype(jnp.bfloat16)
        z1 = dis * jnp.dot(fm1, w1[...], preferred_element_type=_f32)
        y1 = jnp.dot(Ab, z1.astype(jnp.bfloat16),
                     preferred_element_type=_f32) + z1
        x1 = jnp.maximum(dis * y1 + b1[...], 0.0) * mask
        z2 = dis * jnp.dot(x1, w2[...], preferred_element_type=_f32)
        y2 = jnp.dot(Ab, z2.astype(jnp.bfloat16),
                     preferred_element_type=_f32) + z2
        x2 = jnp.maximum(dis * y2 + b2[...], 0.0) * mask
        xs.append(x1)
        xs.append(x2)

    inv = 1.0 / (N * FM)
    att1 = fc1b[...]
    for c in range(6):
        att1 = att1 + (jnp.sum(xs[c]) * inv) * fc1w[c:c + 1, :]
    att1 = jnp.maximum(att1, 0.0)
    att2 = _sigmoid(jnp.dot(att1, fc2w[...],
                            preferred_element_type=_f32) + fc2b[...])
    s = att2 * cw[...]

    acc = jnp.full((NP, FM), cb[0, 0], _f32)
    for c in range(6):
        acc = acc + s[0:1, c:c + 1] * xs[c]
    out_ref[...] = acc


_tc_call = pl.pallas_call(
    _tc_body,
    out_shape=jax.ShapeDtypeStruct((NP, FM), _f32),
)


def _edge_indices(edges, epad, tiles):
    src = edges[0].astype(_i32)
    dst = edges[1].astype(_i32)
    gidx = jnp.concatenate(
        [src * N + dst, jnp.full((epad - E,), GPAD, _i32)])
    sidx = jnp.concatenate(
        [dst * NP + src, SPAD - jnp.arange(epad - E, dtype=_i32)])
    nch = epad // (tiles * CHUNK)
    return (gidx.reshape(tiles, nch, CHUNK), sidx.reshape(tiles, nch, CHUNK))


def kernel(fm1, edges_f, edges_s, edges_g, dm_f, dm_s, dm_g,
           W_x1_f, b_x1_f, W_x2_f, b_x2_f, W_x1_s, b_x1_s, W_x2_s, b_x2_s,
           W_x1_g, b_x1_g, W_x2_g, b_x2_g,
           fc1_W, fc1_b, fc2_W, fc2_b, cnn_w, cnn_b):
    gi_f, si_f = _edge_indices(edges_f, EPADA, 16)
    gi_s, si_s = _edge_indices(edges_s, EPADS, 32)
    gi_g, si_g = _edge_indices(edges_g, EPADA, 16)

    a_f, a_g, a_s0, a_s1 = _sc_build()(
        gi_f, si_f, gi_g, si_g, gi_s, si_s,
        dm_f.reshape(-1), dm_s.reshape(-1), dm_g.reshape(-1))

    fm1p = jnp.pad(fm1, ((0, NP - N), (0, 0)))
    rmask = (jnp.arange(NP) < N).astype(_f32).reshape(NP, 1)
    fc1w = jnp.pad(fc1_W, ((0, 2), (0, 2)))
    fc1b = jnp.pad(fc1_b, (0, 2)).reshape(1, 32)
    fc2w = jnp.pad(fc2_W, ((0, 2), (0, 2)))
    fc2b = jnp.pad(fc2_b, (0, 2)).reshape(1, 8)
    cw = jnp.pad(cnn_w, (0, 2)).reshape(1, 8)
    cb = cnn_b.reshape(1, 1)

    outp = _tc_call(
        fm1p, a_f.reshape(NP, NP), a_s0.reshape(NP, NP),
        a_s1.reshape(NP, NP), a_g.reshape(NP, NP),
        W_x1_f, b_x1_f.reshape(1, FM), W_x2_f, b_x2_f.reshape(1, FM),
        W_x1_s, b_x1_s.reshape(1, FM), W_x2_s, b_x2_s.reshape(1, FM),
        W_x1_g, b_x1_g.reshape(1, FM), W_x2_g, b_x2_g.reshape(1, FM),
        fc1w, fc1b, fc2w, fc2b, cw, cb, rmask)
    return outp[:N]


# speedup vs baseline: 80.5782x; 1.4663x over previous
import functools

import jax
import jax.numpy as jnp
from jax import lax
from jax.experimental import pallas as pl
from jax.experimental.pallas import tpu as pltpu, tpu_sc as plsc

N = 853
FM = 512
E = 54592

NP = 896
NN = NP * NP
CHUNK = 128
NCHA = 27
NCHS = 14
EPADA = 16 * NCHA * CHUNK
EPADS = 32 * NCHS * CHUNK
SLICE = NN // 16
ZCH = SLICE // 16
SPAD = NN - 1
NB = 7
SLAB = NP * CHUNK
RPT = NP // 16

_f32 = jnp.float32
_i32 = jnp.int32


def _sc_body(si_f, si_g, si_s, c_f, c_g, c_s0, c_s1,
             sbA, sbS, ones, zbuf, acc0, acc1):
    cid = lax.axis_index("c")
    sid = lax.axis_index("s")
    base = sid * SLICE

    def run_core(si_a, out_a, out_s):
        w = cid * 16 + sid
        pltpu.sync_copy(si_a.at[sid], sbA)
        pltpu.sync_copy(si_s.at[w], sbS)

        def zero_zbuf(i, c):
            zbuf[pl.ds(i * 16, 16)] = jnp.zeros((16,), _f32)
            return c

        lax.fori_loop(0, ZCH // 16, zero_zbuf, 0)
        for k in range(8):
            ones[pl.ds(k * 16, 16)] = jnp.ones((16,), _f32)
        for k in range(16):
            pltpu.sync_copy(zbuf, acc0.at[pl.ds(base + k * ZCH, ZCH)])
        for k in range(16):
            pltpu.sync_copy(zbuf, acc1.at[pl.ds(base + k * ZCH, ZCH)])
        plsc.subcore_barrier()

        for j in range(NCHA):
            pltpu.sync_copy(ones, acc0.at[sbA.at[j]], add=True)
        for j in range(NCHS):
            pltpu.sync_copy(ones, acc1.at[sbS.at[j]], add=True)
        plsc.subcore_barrier()

        fsl = pl.ds(base, SLICE)
        pltpu.sync_copy(acc0.at[fsl], out_a.at[fsl])
        pltpu.sync_copy(acc1.at[fsl], out_s.at[fsl])

    @pl.when(cid == 0)
    def _():
        run_core(si_f, c_f, c_s0)

    @pl.when(cid == 1)
    def _():
        run_core(si_g, c_g, c_s1)


@functools.cache
def _sc_build():
    return pl.kernel(
        _sc_body,
        out_type=[jax.ShapeDtypeStruct((NN,), _f32)] * 4,
        mesh=plsc.VectorSubcoreMesh(core_axis_name="c", subcore_axis_name="s"),
        scratch_types=[
            pltpu.VMEM((NCHA, CHUNK), _i32),
            pltpu.VMEM((NCHS, CHUNK), _i32),
            pltpu.VMEM((CHUNK,), _f32),
            pltpu.VMEM((ZCH,), _f32),
            pltpu.VMEM_SHARED((NN,), _f32),
            pltpu.VMEM_SHARED((NN,), _f32),
        ],
    )


def _sigmoid(x):
    return 1.0 / (1.0 + jnp.exp(-x))


def _tc_body(fm1_ref, dmf_ref, dms_ref, dmg_ref,
             cf_ref, cs0_ref, cs1_ref, cg_ref,
             w1f, b1f, w2f, b2f, w1s, b1s, w2s, b2s, w1g, b1g, w2g, b2g,
             fc1w, fc1b, fc2w, fc2b, cw, cb, rmask, out_ref):
    fm1 = fm1_ref[...]
    mask = rmask[...]
    xs = []
    for cnt_ref, dm_ref, w1, b1, w2, b2 in (
            (cf_ref, dmf_ref, w1f, b1f, w2f, b2f),
            ((cs0_ref, cs1_ref), dms_ref, w1s, b1s, w2s, b2s),
            (cg_ref, dmg_ref, w1g, b1g, w2g, b2g)):
        if isinstance(cnt_ref, tuple):
            C = cnt_ref[0][...] + cnt_ref[1][...]
        else:
            C = cnt_ref[...]
        blocks = []
        deg = jnp.ones((NP, 1), _f32)
        for c in range(NB):
            dmt = jnp.swapaxes(dm_ref[c * CHUNK:(c + 1) * CHUNK, :], 0, 1)
            blk = C[c] * dmt
            deg = deg + jnp.sum(blk, axis=1, keepdims=True)
            blocks.append(blk.astype(jnp.bfloat16))
        dis = jnp.where(deg > 0, lax.rsqrt(deg), 0.0) * mask

        z1 = dis * jnp.dot(fm1, w1[...], preferred_element_type=_f32)
        z1b = z1.astype(jnp.bfloat16)
        y1 = z1
        for c in range(NB):
            y1 = y1 + jnp.dot(blocks[c], z1b[c * CHUNK:(c + 1) * CHUNK, :],
                              preferred_element_type=_f32)
        x1 = jnp.maximum(dis * y1 + b1[...], 0.0) * mask
        z2 = dis * jnp.dot(x1, w2[...], preferred_element_type=_f32)
        z2b = z2.astype(jnp.bfloat16)
        y2 = z2
        for c in range(NB):
            y2 = y2 + jnp.dot(blocks[c], z2b[c * CHUNK:(c + 1) * CHUNK, :],
                              preferred_element_type=_f32)
        x2 = jnp.maximum(dis * y2 + b2[...], 0.0) * mask
        xs.append(x1)
        xs.append(x2)

    inv = 1.0 / (N * FM)
    att1 = fc1b[...]
    for c in range(6):
        att1 = att1 + (jnp.sum(xs[c]) * inv) * fc1w[c:c + 1, :]
    att1 = jnp.maximum(att1, 0.0)
    att2 = _sigmoid(jnp.dot(att1, fc2w[...],
                            preferred_element_type=_f32) + fc2b[...])
    s = att2 * cw[...]

    acc = jnp.full((NP, FM), cb[0, 0], _f32)
    for c in range(6):
        acc = acc + s[0:1, c:c + 1] * xs[c]
    out_ref[...] = acc


_tc_call = pl.pallas_call(
    _tc_body,
    out_shape=jax.ShapeDtypeStruct((NP, FM), _f32),
)


def _edge_indices(edges, epad, tiles):
    src = edges[0].astype(_i32)
    dst = edges[1].astype(_i32)
    sidx = (src // CHUNK) * SLAB + dst * CHUNK + (src % CHUNK)
    sidx = jnp.concatenate(
        [sidx, SPAD - jnp.arange(epad - E, dtype=_i32)])
    nch = epad // (tiles * CHUNK)
    return sidx.reshape(tiles, nch, CHUNK)


def kernel(fm1, edges_f, edges_s, edges_g, dm_f, dm_s, dm_g,
           W_x1_f, b_x1_f, W_x2_f, b_x2_f, W_x1_s, b_x1_s, W_x2_s, b_x2_s,
           W_x1_g, b_x1_g, W_x2_g, b_x2_g,
           fc1_W, fc1_b, fc2_W, fc2_b, cnn_w, cnn_b):
    si_f = _edge_indices(edges_f, EPADA, 16)
    si_s = _edge_indices(edges_s, EPADS, 32)
    si_g = _edge_indices(edges_g, EPADA, 16)

    c_f, c_g, c_s0, c_s1 = [
        a.reshape(NB, NP, CHUNK) for a in _sc_build()(si_f, si_g, si_s)]

    fm1p = jnp.pad(fm1, ((0, NP - N), (0, 0)))
    dmfp = jnp.pad(dm_f, ((0, NP - N), (0, NP - N)))
    dmsp = jnp.pad(dm_s, ((0, NP - N), (0, NP - N)))
    dmgp = jnp.pad(dm_g, ((0, NP - N), (0, NP - N)))
    rmask = (jnp.arange(NP) < N).astype(_f32).reshape(NP, 1)
    fc1w = jnp.pad(fc1_W, ((0, 2), (0, 2)))
    fc1b = jnp.pad(fc1_b, (0, 2)).reshape(1, 32)
    fc2w = jnp.pad(fc2_W, ((0, 2), (0, 2)))
    fc2b = jnp.pad(fc2_b, (0, 2)).reshape(1, 8)
    cw = jnp.pad(cnn_w, (0, 2)).reshape(1, 8)
    cb = cnn_b.reshape(1, 1)

    outp = _tc_call(
        fm1p, dmfp, dmsp, dmgp, c_f, c_s0, c_s1, c_g,
        W_x1_f, b_x1_f.reshape(1, FM), W_x2_f, b_x2_f.reshape(1, FM),
        W_x1_s, b_x1_s.reshape(1, FM), W_x2_s, b_x2_s.reshape(1, FM),
        W_x1_g, b_x1_g.reshape(1, FM), W_x2_g, b_x2_g.reshape(1, FM),
        fc1w, fc1b, fc2w, fc2b, cw, cb, rmask)
    return outp[:N]


# speedup vs baseline: 87.6268x; 1.0875x over previous
import functools

import jax
import jax.numpy as jnp
from jax import lax
from jax.experimental import pallas as pl
from jax.experimental.pallas import tpu as pltpu, tpu_sc as plsc

N = 853
FM = 512
E = 54592

NP = 896
NN = NP * NP
CHUNK = 128
NCHA = 27
NCHS = 14
EPADA = 16 * NCHA * CHUNK
EPADS = 32 * NCHS * CHUNK
SLICE = NN // 16
ZCH = SLICE // 16
SPAD = NN - 1
NB = 7
SLAB = NP * CHUNK
RPT = NP // 16

_f32 = jnp.float32
_i32 = jnp.int32


def _sc_body(si_f, si_g, si_s, c_f, c_g, c_s0, c_s1,
             sbA, sbS, ones, zbuf, acc0, acc1):
    cid = lax.axis_index("c")
    sid = lax.axis_index("s")
    base = sid * SLICE

    def run_core(si_a, out_a, out_s):
        w = cid * 16 + sid
        pltpu.sync_copy(si_a.at[sid], sbA)
        pltpu.sync_copy(si_s.at[w], sbS)

        def zero_zbuf(i, c):
            zbuf[pl.ds(i * 16, 16)] = jnp.zeros((16,), _f32)
            return c

        lax.fori_loop(0, ZCH // 16, zero_zbuf, 0)
        for k in range(8):
            ones[pl.ds(k * 16, 16)] = jnp.ones((16,), _f32)
        for k in range(16):
            pltpu.sync_copy(zbuf, acc0.at[pl.ds(base + k * ZCH, ZCH)])
        for k in range(16):
            pltpu.sync_copy(zbuf, acc1.at[pl.ds(base + k * ZCH, ZCH)])
        plsc.subcore_barrier()

        for j in range(NCHA):
            pltpu.sync_copy(ones, acc0.at[sbA.at[j]], add=True)
        for j in range(NCHS):
            pltpu.sync_copy(ones, acc1.at[sbS.at[j]], add=True)
        plsc.subcore_barrier()

        fsl = pl.ds(base, SLICE)
        pltpu.sync_copy(acc0.at[fsl], out_a.at[fsl])
        pltpu.sync_copy(acc1.at[fsl], out_s.at[fsl])

    @pl.when(cid == 0)
    def _():
        run_core(si_f, c_f, c_s0)

    @pl.when(cid == 1)
    def _():
        run_core(si_g, c_g, c_s1)


@functools.cache
def _sc_build():
    return pl.kernel(
        _sc_body,
        out_type=[jax.ShapeDtypeStruct((NN,), _f32)] * 4,
        mesh=plsc.VectorSubcoreMesh(core_axis_name="c", subcore_axis_name="s"),
        scratch_types=[
            pltpu.VMEM((NCHA, CHUNK), _i32),
            pltpu.VMEM((NCHS, CHUNK), _i32),
            pltpu.VMEM((CHUNK,), _f32),
            pltpu.VMEM((ZCH,), _f32),
            pltpu.VMEM_SHARED((NN,), _f32),
            pltpu.VMEM_SHARED((NN,), _f32),
        ],
    )


def _sigmoid(x):
    return 1.0 / (1.0 + jnp.exp(-x))


def _tc_body(fm1_ref, dmf_ref, dms_ref, dmg_ref,
             cf_ref, cs0_ref, cs1_ref, cg_ref,
             w1f, b1f, w2f, b2f, w1s, b1s, w2s, b2s, w1g, b1g, w2g, b2g,
             fc1w, fc1b, fc2w, fc2b, cw, cb, rmask, out_ref):
    fm1 = fm1_ref[...]
    mask = rmask[...]
    xs = []
    for cnt_ref, dm_ref, w1, b1, w2, b2 in (
            (cf_ref, dmf_ref, w1f, b1f, w2f, b2f),
            ((cs0_ref, cs1_ref), dms_ref, w1s, b1s, w2s, b2s),
            (cg_ref, dmg_ref, w1g, b1g, w2g, b2g)):
        if isinstance(cnt_ref, tuple):
            C = cnt_ref[0][...] + cnt_ref[1][...]
        else:
            C = cnt_ref[...]
        blocks = []
        deg = jnp.ones((NP, 1), _f32)
        for c in range(NB):
            dmt = jnp.swapaxes(dm_ref[c * CHUNK:(c + 1) * CHUNK, :], 0, 1)
            blk = C[c].astype(jnp.bfloat16) * dmt
            deg = deg + jnp.sum(blk.astype(_f32), axis=1, keepdims=True)
            blocks.append(blk)
        dis = jnp.where(deg > 0, lax.rsqrt(deg), 0.0) * mask

        z1 = dis * jnp.dot(fm1, w1[...], preferred_element_type=_f32)
        z1b = z1.astype(jnp.bfloat16)
        y1 = z1
        for c in range(NB):
            y1 = y1 + jnp.dot(blocks[c], z1b[c * CHUNK:(c + 1) * CHUNK, :],
                              preferred_element_type=_f32)
        x1 = jnp.maximum(dis * y1 + b1[...], 0.0) * mask
        z2 = dis * jnp.dot(x1.astype(jnp.bfloat16), w2[...],
                           preferred_element_type=_f32)
        z2b = z2.astype(jnp.bfloat16)
        y2 = z2
        for c in range(NB):
            y2 = y2 + jnp.dot(blocks[c], z2b[c * CHUNK:(c + 1) * CHUNK, :],
                              preferred_element_type=_f32)
        x2 = jnp.maximum(dis * y2 + b2[...], 0.0) * mask
        xs.append(x1)
        xs.append(x2)

    inv = 1.0 / (N * FM)
    att1 = fc1b[...]
    for c in range(6):
        att1 = att1 + (jnp.sum(xs[c]) * inv) * fc1w[c:c + 1, :]
    att1 = jnp.maximum(att1, 0.0)
    att2 = _sigmoid(jnp.dot(att1, fc2w[...],
                            preferred_element_type=_f32) + fc2b[...])
    s = att2 * cw[...]

    acc = jnp.full((NP, FM), cb[0, 0], _f32)
    for c in range(6):
        acc = acc + s[0:1, c:c + 1] * xs[c]
    out_ref[...] = acc[:N]


_tc_call = pl.pallas_call(
    _tc_body,
    out_shape=jax.ShapeDtypeStruct((N, FM), _f32),
)


def _edge_indices(edges, epad, tiles):
    src = edges[0].astype(_i32)
    dst = edges[1].astype(_i32)
    sidx = (src // CHUNK) * SLAB + dst * CHUNK + (src % CHUNK)
    sidx = jnp.concatenate(
        [sidx, SPAD - jnp.arange(epad - E, dtype=_i32)])
    nch = epad // (tiles * CHUNK)
    return sidx.reshape(tiles, nch, CHUNK)


def kernel(fm1, edges_f, edges_s, edges_g, dm_f, dm_s, dm_g,
           W_x1_f, b_x1_f, W_x2_f, b_x2_f, W_x1_s, b_x1_s, W_x2_s, b_x2_s,
           W_x1_g, b_x1_g, W_x2_g, b_x2_g,
           fc1_W, fc1_b, fc2_W, fc2_b, cnn_w, cnn_b):
    si_f = _edge_indices(edges_f, EPADA, 16)
    si_s = _edge_indices(edges_s, EPADS, 32)
    si_g = _edge_indices(edges_g, EPADA, 16)

    c_f, c_g, c_s0, c_s1 = [
        a.reshape(NB, NP, CHUNK) for a in _sc_build()(si_f, si_g, si_s)]

    _bf = jnp.bfloat16
    fm1p = jnp.pad(fm1.astype(_bf), ((0, NP - N), (0, 0)))
    dmfp = jnp.pad(dm_f.astype(_bf), ((0, NP - N), (0, NP - N)))
    dmsp = jnp.pad(dm_s.astype(_bf), ((0, NP - N), (0, NP - N)))
    dmgp = jnp.pad(dm_g.astype(_bf), ((0, NP - N), (0, NP - N)))
    rmask = (jnp.arange(NP) < N).astype(_f32).reshape(NP, 1)
    fc1w = jnp.pad(fc1_W, ((0, 2), (0, 2)))
    fc1b = jnp.pad(fc1_b, (0, 2)).reshape(1, 32)
    fc2w = jnp.pad(fc2_W, ((0, 2), (0, 2)))
    fc2b = jnp.pad(fc2_b, (0, 2)).reshape(1, 8)
    cw = jnp.pad(cnn_w, (0, 2)).reshape(1, 8)
    cb = cnn_b.reshape(1, 1)

    return _tc_call(
        fm1p, dmfp, dmsp, dmgp, c_f, c_s0, c_s1, c_g,
        W_x1_f.astype(_bf), b_x1_f.reshape(1, FM),
        W_x2_f.astype(_bf), b_x2_f.reshape(1, FM),
        W_x1_s.astype(_bf), b_x1_s.reshape(1, FM),
        W_x2_s.astype(_bf), b_x2_s.reshape(1, FM),
        W_x1_g.astype(_bf), b_x1_g.reshape(1, FM),
        W_x2_g.astype(_bf), b_x2_g.reshape(1, FM),
        fc1w, fc1b, fc2w, fc2b, cw, cb, rmask)


# speedup vs baseline: 88.5465x; 1.0105x over previous
import functools

import jax
import jax.numpy as jnp
from jax import lax
from jax.experimental import pallas as pl
from jax.experimental.pallas import tpu as pltpu, tpu_sc as plsc

N = 853
FM = 512
E = 54592

NP = 896
NN = NP * NP
CHUNK = 128
NCHA = 27
NCHS = 14
EPADA = 16 * NCHA * CHUNK
EPADS = 32 * NCHS * CHUNK
SLICE = NN // 16
ZCH = SLICE // 16
SPAD = NN - 1
NB = 7
SLAB = NP * CHUNK
RPT = NP // 16

_f32 = jnp.float32
_i32 = jnp.int32


def _sc_body(si_f, si_g, si_s, c_f, c_g, c_s0, c_s1,
             sbA, sbS, ones, zbuf, acc0, acc1, sem):
    cid = lax.axis_index("c")
    sid = lax.axis_index("s")
    base = sid * SLICE

    def run_core(si_a, out_a, out_s):
        w = cid * 16 + sid
        idx_cps = [pltpu.async_copy(si_a.at[sid], sbA, sem),
                   pltpu.async_copy(si_s.at[w], sbS, sem)]

        def zero_zbuf(i, c):
            zbuf[pl.ds(i * 16, 16)] = jnp.zeros((16,), _f32)
            return c

        lax.fori_loop(0, ZCH // 16, zero_zbuf, 0)
        for k in range(8):
            ones[pl.ds(k * 16, 16)] = jnp.ones((16,), _f32)

        zero_cps = []
        for k in range(16):
            zero_cps.append(pltpu.async_copy(
                zbuf, acc0.at[pl.ds(base + k * ZCH, ZCH)], sem))
        for k in range(16):
            zero_cps.append(pltpu.async_copy(
                zbuf, acc1.at[pl.ds(base + k * ZCH, ZCH)], sem))
        for c in idx_cps:
            c.wait()
        for c in zero_cps:
            c.wait()
        plsc.subcore_barrier()

        sc_cps = [pltpu.async_copy(ones, acc0.at[sbA.at[j]], sem, add=True)
                  for j in range(NCHA)]
        sc_cps += [pltpu.async_copy(ones, acc1.at[sbS.at[j]], sem, add=True)
                   for j in range(NCHS)]
        for c in sc_cps:
            c.wait()
        plsc.subcore_barrier()

        fsl = pl.ds(base, SLICE)
        out_cps = [pltpu.async_copy(acc0.at[fsl], out_a.at[fsl], sem),
                   pltpu.async_copy(acc1.at[fsl], out_s.at[fsl], sem)]
        for c in out_cps:
            c.wait()

    @pl.when(cid == 0)
    def _():
        run_core(si_f, c_f, c_s0)

    @pl.when(cid == 1)
    def _():
        run_core(si_g, c_g, c_s1)


@functools.cache
def _sc_build():
    return pl.kernel(
        _sc_body,
        out_type=[jax.ShapeDtypeStruct((NN,), _f32)] * 4,
        mesh=plsc.VectorSubcoreMesh(core_axis_name="c", subcore_axis_name="s"),
        scratch_types=[
            pltpu.VMEM((NCHA, CHUNK), _i32),
            pltpu.VMEM((NCHS, CHUNK), _i32),
            pltpu.VMEM((CHUNK,), _f32),
            pltpu.VMEM((ZCH,), _f32),
            pltpu.VMEM_SHARED((NN,), _f32),
            pltpu.VMEM_SHARED((NN,), _f32),
            pltpu.SemaphoreType.DMA,
        ],
    )


def _sigmoid(x):
    return 1.0 / (1.0 + jnp.exp(-x))


def _tc_body(fm1_ref, dmf_ref, dms_ref, dmg_ref,
             cf_ref, cs0_ref, cs1_ref, cg_ref,
             w1f, b1f, w2f, b2f, w1s, b1s, w2s, b2s, w1g, b1g, w2g, b2g,
             fc1w, fc1b, fc2w, fc2b, cw, cb, rmask, out_ref):
    fm1 = fm1_ref[...]
    mask = rmask[...]
    xs = []
    for cnt_ref, dm_ref, w1, b1, w2, b2 in (
            (cf_ref, dmf_ref, w1f, b1f, w2f, b2f),
            ((cs0_ref, cs1_ref), dms_ref, w1s, b1s, w2s, b2s),
            (cg_ref, dmg_ref, w1g, b1g, w2g, b2g)):
        if isinstance(cnt_ref, tuple):
            C = cnt_ref[0][...] + cnt_ref[1][...]
        else:
            C = cnt_ref[...]
        blocks = []
        deg = jnp.ones((NP, 1), _f32)
        for c in range(NB):
            dmt = jnp.swapaxes(dm_ref[c * CHUNK:(c + 1) * CHUNK, :], 0, 1)
            blk = C[c].astype(jnp.bfloat16) * dmt
            deg = deg + jnp.sum(blk.astype(_f32), axis=1, keepdims=True)
            blocks.append(blk)
        dis = jnp.where(deg > 0, lax.rsqrt(deg), 0.0) * mask

        z1 = dis * jnp.dot(fm1, w1[...], preferred_element_type=_f32)
        z1b = z1.astype(jnp.bfloat16)
        y1 = z1
        for c in range(NB):
            y1 = y1 + jnp.dot(blocks[c], z1b[c * CHUNK:(c + 1) * CHUNK, :],
                              preferred_element_type=_f32)
        x1 = jnp.maximum(dis * y1 + b1[...], 0.0) * mask
        z2 = dis * jnp.dot(x1.astype(jnp.bfloat16), w2[...],
                           preferred_element_type=_f32)
        z2b = z2.astype(jnp.bfloat16)
        y2 = z2
        for c in range(NB):
            y2 = y2 + jnp.dot(blocks[c], z2b[c * CHUNK:(c + 1) * CHUNK, :],
                              preferred_element_type=_f32)
        x2 = jnp.maximum(dis * y2 + b2[...], 0.0) * mask
        xs.append(x1)
        xs.append(x2)

    inv = 1.0 / (N * FM)
    att1 = fc1b[...]
    for c in range(6):
        att1 = att1 + (jnp.sum(xs[c]) * inv) * fc1w[c:c + 1, :]
    att1 = jnp.maximum(att1, 0.0)
    att2 = _sigmoid(jnp.dot(att1, fc2w[...],
                            preferred_element_type=_f32) + fc2b[...])
    s = att2 * cw[...]

    acc = jnp.full((NP, FM), cb[0, 0], _f32)
    for c in range(6):
        acc = acc + s[0:1, c:c + 1] * xs[c]
    out_ref[...] = acc[:N]


_tc_call = pl.pallas_call(
    _tc_body,
    out_shape=jax.ShapeDtypeStruct((N, FM), _f32),
)


def _edge_indices(edges, epad, tiles):
    src = edges[0].astype(_i32)
    dst = edges[1].astype(_i32)
    sidx = (src // CHUNK) * SLAB + dst * CHUNK + (src % CHUNK)
    sidx = jnp.concatenate(
        [sidx, SPAD - jnp.arange(epad - E, dtype=_i32)])
    nch = epad // (tiles * CHUNK)
    return sidx.reshape(tiles, nch, CHUNK)


def kernel(fm1, edges_f, edges_s, edges_g, dm_f, dm_s, dm_g,
           W_x1_f, b_x1_f, W_x2_f, b_x2_f, W_x1_s, b_x1_s, W_x2_s, b_x2_s,
           W_x1_g, b_x1_g, W_x2_g, b_x2_g,
           fc1_W, fc1_b, fc2_W, fc2_b, cnn_w, cnn_b):
    si_f = _edge_indices(edges_f, EPADA, 16)
    si_s = _edge_indices(edges_s, EPADS, 32)
    si_g = _edge_indices(edges_g, EPADA, 16)

    c_f, c_g, c_s0, c_s1 = [
        a.reshape(NB, NP, CHUNK) for a in _sc_build()(si_f, si_g, si_s)]

    _bf = jnp.bfloat16
    fm1p = jnp.pad(fm1.astype(_bf), ((0, NP - N), (0, 0)))
    dmfp = jnp.pad(dm_f.astype(_bf), ((0, NP - N), (0, NP - N)))
    dmsp = jnp.pad(dm_s.astype(_bf), ((0, NP - N), (0, NP - N)))
    dmgp = jnp.pad(dm_g.astype(_bf), ((0, NP - N), (0, NP - N)))
    rmask = (jnp.arange(NP) < N).astype(_f32).reshape(NP, 1)
    fc1w = jnp.pad(fc1_W, ((0, 2), (0, 2)))
    fc1b = jnp.pad(fc1_b, (0, 2)).reshape(1, 32)
    fc2w = jnp.pad(fc2_W, ((0, 2), (0, 2)))
    fc2b = jnp.pad(fc2_b, (0, 2)).reshape(1, 8)
    cw = jnp.pad(cnn_w, (0, 2)).reshape(1, 8)
    cb = cnn_b.reshape(1, 1)

    return _tc_call(
        fm1p, dmfp, dmsp, dmgp, c_f, c_s0, c_s1, c_g,
        W_x1_f.astype(_bf), b_x1_f.reshape(1, FM),
        W_x2_f.astype(_bf), b_x2_f.reshape(1, FM),
        W_x1_s.astype(_bf), b_x1_s.reshape(1, FM),
        W_x2_s.astype(_bf), b_x2_s.reshape(1, FM),
        W_x1_g.astype(_bf), b_x1_g.reshape(1, FM),
        W_x2_g.astype(_bf), b_x2_g.reshape(1, FM),
        fc1w, fc1b, fc2w, fc2b, cw, cb, rmask)


# speedup vs baseline: 89.8739x; 1.0150x over previous
import functools

import jax
import jax.numpy as jnp
from jax import lax
from jax.experimental import pallas as pl
from jax.experimental.pallas import tpu as pltpu, tpu_sc as plsc

N = 853
FM = 512
E = 54592

NP = 896
NN = NP * NP
CHUNK = 128
NCHA = 32
NCHS = 16
EPADA = 16 * NCHA * CHUNK
EPADS = 32 * NCHS * CHUNK
DEAD = (NP - N) * CHUNK
SLICE = NN // 16
ZCH = SLICE // 16
SPAD = NN - 1
NB = 7
SLAB = NP * CHUNK
RPT = NP // 16

_f32 = jnp.float32
_i32 = jnp.int32


def _sc_body(si_f, si_g, si_s, c_f, c_g, c_s0, c_s1,
             sbA, sbS, ones, zbuf, acc0, acc1, sem):
    cid = lax.axis_index("c")
    sid = lax.axis_index("s")
    base = sid * SLICE

    def run_core(si_a, out_a, out_s):
        w = cid * 16 + sid
        idx_cps = [pltpu.async_copy(si_a.at[sid], sbA, sem),
                   pltpu.async_copy(si_s.at[w], sbS, sem)]

        def zero_zbuf(i, c):
            zbuf[pl.ds(i * 16, 16)] = jnp.zeros((16,), _f32)
            return c

        lax.fori_loop(0, ZCH // 16, zero_zbuf, 0)
        for k in range(8):
            ones[pl.ds(k * 16, 16)] = jnp.ones((16,), _f32)

        zero_cps = []
        for k in range(16):
            zero_cps.append(pltpu.async_copy(
                zbuf, acc0.at[pl.ds(base + k * ZCH, ZCH)], sem))
        for k in range(16):
            zero_cps.append(pltpu.async_copy(
                zbuf, acc1.at[pl.ds(base + k * ZCH, ZCH)], sem))
        for c in idx_cps:
            c.wait()
        for c in zero_cps:
            c.wait()
        plsc.subcore_barrier()

        sc_cps = [pltpu.async_copy(ones, acc0.at[sbA.at[j]], sem, add=True)
                  for j in range(NCHA)]
        sc_cps += [pltpu.async_copy(ones, acc1.at[sbS.at[j]], sem, add=True)
                   for j in range(NCHS)]
        for c in sc_cps:
            c.wait()
        plsc.subcore_barrier()

        fsl = pl.ds(base, SLICE)
        out_cps = [pltpu.async_copy(acc0.at[fsl], out_a.at[fsl], sem),
                   pltpu.async_copy(acc1.at[fsl], out_s.at[fsl], sem)]
        for c in out_cps:
            c.wait()

    @pl.when(cid == 0)
    def _():
        run_core(si_f, c_f, c_s0)

    @pl.when(cid == 1)
    def _():
        run_core(si_g, c_g, c_s1)


@functools.cache
def _sc_build():
    return pl.kernel(
        _sc_body,
        out_type=[jax.ShapeDtypeStruct((NN,), _f32)] * 4,
        mesh=plsc.VectorSubcoreMesh(core_axis_name="c", subcore_axis_name="s"),
        scratch_types=[
            pltpu.VMEM((NCHA, CHUNK), _i32),
            pltpu.VMEM((NCHS, CHUNK), _i32),
            pltpu.VMEM((CHUNK,), _f32),
            pltpu.VMEM((ZCH,), _f32),
            pltpu.VMEM_SHARED((NN,), _f32),
            pltpu.VMEM_SHARED((NN,), _f32),
            pltpu.SemaphoreType.DMA,
        ],
    )


def _sigmoid(x):
    return 1.0 / (1.0 + jnp.exp(-x))


def _tc_body(fm1_ref, dmf_ref, dms_ref, dmg_ref,
             cf_ref, cs0_ref, cs1_ref, cg_ref,
             w1f, b1f, w2f, b2f, w1s, b1s, w2s, b2s, w1g, b1g, w2g, b2g,
             fc1w, fc1b, fc2w, fc2b, cw, cb, rmask, out_ref):
    fm1 = fm1_ref[...]
    mask = rmask[...]
    xs = []
    for cnt_ref, dm_ref, w1, b1, w2, b2 in (
            (cf_ref, dmf_ref, w1f, b1f, w2f, b2f),
            ((cs0_ref, cs1_ref), dms_ref, w1s, b1s, w2s, b2s),
            (cg_ref, dmg_ref, w1g, b1g, w2g, b2g)):
        if isinstance(cnt_ref, tuple):
            C = cnt_ref[0][...] + cnt_ref[1][...]
        else:
            C = cnt_ref[...]
        blocks = []
        deg = jnp.ones((NP, 1), _f32)
        for c in range(NB):
            dmt = jnp.swapaxes(dm_ref[c * CHUNK:(c + 1) * CHUNK, :], 0, 1)
            blk = C[c].astype(jnp.bfloat16) * dmt
            deg = deg + jnp.sum(blk.astype(_f32), axis=1, keepdims=True)
            blocks.append(blk)
        dis = jnp.where(deg > 0, lax.rsqrt(deg), 0.0) * mask

        z1 = dis * jnp.dot(fm1, w1[...], preferred_element_type=_f32)
        z1b = z1.astype(jnp.bfloat16)
        y1 = z1
        for c in range(NB):
            y1 = y1 + jnp.dot(blocks[c], z1b[c * CHUNK:(c + 1) * CHUNK, :],
                              preferred_element_type=_f32)
        x1 = jnp.maximum(dis * y1 + b1[...], 0.0) * mask
        z2 = dis * jnp.dot(x1.astype(jnp.bfloat16), w2[...],
                           preferred_element_type=_f32)
        z2b = z2.astype(jnp.bfloat16)
        y2 = z2
        for c in range(NB):
            y2 = y2 + jnp.dot(blocks[c], z2b[c * CHUNK:(c + 1) * CHUNK, :],
                              preferred_element_type=_f32)
        x2 = jnp.maximum(dis * y2 + b2[...], 0.0) * mask
        xs.append(x1)
        xs.append(x2)

    inv = 1.0 / (N * FM)
    att1 = fc1b[...]
    for c in range(6):
        att1 = att1 + (jnp.sum(xs[c]) * inv) * fc1w[c:c + 1, :]
    att1 = jnp.maximum(att1, 0.0)
    att2 = _sigmoid(jnp.dot(att1, fc2w[...],
                            preferred_element_type=_f32) + fc2b[...])
    s = att2 * cw[...]

    acc = jnp.full((NP, FM), cb[0, 0], _f32)
    for c in range(6):
        acc = acc + s[0:1, c:c + 1] * xs[c]
    out_ref[...] = acc[:N]


_tc_call = pl.pallas_call(
    _tc_body,
    out_shape=jax.ShapeDtypeStruct((N, FM), _f32),
)


def _edge_indices(edges, epad, tiles):
    src = edges[0].astype(_i32)
    dst = edges[1].astype(_i32)
    sidx = (src // CHUNK) * SLAB + dst * CHUNK + (src % CHUNK)
    k = jnp.arange(epad - E, dtype=_i32)
    dead = (NB - 1 - k // DEAD) * SLAB + N * CHUNK + (k % DEAD)
    sidx = jnp.concatenate([sidx, dead])
    nch = epad // (tiles * CHUNK)
    return sidx.reshape(tiles, nch, CHUNK)


def kernel(fm1, edges_f, edges_s, edges_g, dm_f, dm_s, dm_g,
           W_x1_f, b_x1_f, W_x2_f, b_x2_f, W_x1_s, b_x1_s, W_x2_s, b_x2_s,
           W_x1_g, b_x1_g, W_x2_g, b_x2_g,
           fc1_W, fc1_b, fc2_W, fc2_b, cnn_w, cnn_b):
    si_f = _edge_indices(edges_f, EPADA, 16)
    si_s = _edge_indices(edges_s, EPADS, 32)
    si_g = _edge_indices(edges_g, EPADA, 16)

    c_f, c_g, c_s0, c_s1 = [
        a.reshape(NB, NP, CHUNK) for a in _sc_build()(si_f, si_g, si_s)]

    _bf = jnp.bfloat16
    fm1p = jnp.pad(fm1.astype(_bf), ((0, NP - N), (0, 0)))
    dmfp = jnp.pad(dm_f.astype(_bf), ((0, NP - N), (0, NP - N)))
    dmsp = jnp.pad(dm_s.astype(_bf), ((0, NP - N), (0, NP - N)))
    dmgp = jnp.pad(dm_g.astype(_bf), ((0, NP - N), (0, NP - N)))
    rmask = (jnp.arange(NP) < N).astype(_f32).reshape(NP, 1)
    fc1w = jnp.pad(fc1_W, ((0, 2), (0, 2)))
    fc1b = jnp.pad(fc1_b, (0, 2)).reshape(1, 32)
    fc2w = jnp.pad(fc2_W, ((0, 2), (0, 2)))
    fc2b = jnp.pad(fc2_b, (0, 2)).reshape(1, 8)
    cw = jnp.pad(cnn_w, (0, 2)).reshape(1, 8)
    cb = cnn_b.reshape(1, 1)

    return _tc_call(
        fm1p, dmfp, dmsp, dmgp, c_f, c_s0, c_s1, c_g,
        W_x1_f.astype(_bf), b_x1_f.reshape(1, FM),
        W_x2_f.astype(_bf), b_x2_f.reshape(1, FM),
        W_x1_s.astype(_bf), b_x1_s.reshape(1, FM),
        W_x2_s.astype(_bf), b_x2_s.reshape(1, FM),
        W_x1_g.astype(_bf), b_x1_g.reshape(1, FM),
        W_x2_g.astype(_bf), b_x2_g.reshape(1, FM),
        fc1w, fc1b, fc2w, fc2b, cw, cb, rmask)
